# baseline (speedup 1.0000x reference)
import functools

import jax
import jax.numpy as jnp
from jax import lax
from jax.experimental import pallas as pl
from jax.experimental.pallas import tpu as pltpu


def _roundup(n, m):
    return ((n + m - 1) // m) * m


def _se_body(x_ref, w1_ref, w2t_ref, o_ref, *, inv_hw):
    xv = x_ref[...]

    pooled = jnp.sum(xv, axis=1, dtype=jnp.float32) * inv_hw

    h = lax.dot_general(
        pooled.astype(w1_ref.dtype), w1_ref[...],
        dimension_numbers=(((1,), (1,)), ((), ())),
        preferred_element_type=jnp.float32,
        precision=lax.Precision.HIGHEST)
    h = jnp.where(h >= 0, h, 0.01 * h)
    s = lax.dot_general(
        h.astype(w2t_ref.dtype), w2t_ref[...],
        dimension_numbers=(((1,), (0,)), ((), ())),
        preferred_element_type=jnp.float32,
        precision=lax.Precision.HIGHEST)
    gate = jax.nn.sigmoid(s).astype(o_ref.dtype)

    o_ref[...] = xv * gate[:, None, :]


def _pick_batch_tile(B, bytes_per_image, budget_bytes):
    best = 1
    for bt in range(1, B + 1):
        if B % bt:
            continue
        steps = B // bt
        if steps % 2 and steps != 1:
            continue
        if 4 * bt * bytes_per_image > budget_bytes:
            break
        best = bt
    return best


def kernel(x, w1, w2):
    B, C, H, W = x.shape
    Cr = w1.shape[0]
    HW = H * W
    xt = jnp.transpose(x, (0, 2, 3, 1)).reshape(B, HW, C)

    itemsize = jnp.dtype(x.dtype).itemsize
    sub = 8 * max(1, 4 // itemsize)
    bytes_per_image = _roundup(HW, sub) * _roundup(C, 128) * itemsize

    budget = 56 << 20
    Bt = _pick_batch_tile(B, bytes_per_image, budget)

    out_t = pl.pallas_call(
        functools.partial(_se_body, inv_hw=1.0 / HW),
        out_shape=jax.ShapeDtypeStruct((B, HW, C), x.dtype),
        grid=(B // Bt,),
        in_specs=[
            pl.BlockSpec((Bt, HW, C), lambda b: (b, 0, 0)),
            pl.BlockSpec((Cr, C), lambda b: (0, 0)),
            pl.BlockSpec((Cr, C), lambda b: (0, 0)),
        ],
        out_specs=pl.BlockSpec((Bt, HW, C), lambda b: (b, 0, 0)),
        compiler_params=pltpu.CompilerParams(
            dimension_semantics=("parallel",),
            vmem_limit_bytes=(62 << 20)),
    )(xt, w1, w2.T)
    return jnp.transpose(out_t.reshape(B, H, W, C), (0, 3, 1, 2))
